# baseline (speedup 1.0000x reference)
import functools

import jax
import jax.numpy as jnp
from jax import lax
from jax.experimental import pallas as pl
from jax.experimental.pallas import tpu as pltpu
from jax.experimental.pallas import tpu_sc as plsc

B = 4096
F = 26
FH = F // 2
V = 100000
D = 32
NN = 13

NC = 2
NS = 16
NW = NC * NS
BPW = B // NW
L = 16
NCH = BPW // L


def _sc_body(idx_hbm, idxf_hbm, int_hbm, lin_hbm, s_out, q_out, l_out,
             idx_v, idxf_v, cols_v, lin_v, s_v, q_v, l_v, sem_c, sem_l):
    wid = lax.axis_index("s") * NC + lax.axis_index("c")
    base = wid * BPW
    pltpu.sync_copy(idx_hbm.at[wid], idx_v)
    pltpu.sync_copy(idxf_hbm.at[wid], idxf_v)

    def fire(f, carry):
        for d in range(D):
            pltpu.async_copy(int_hbm.at[f, d].at[idx_v.at[f]],
                             cols_v.at[f, d], sem_c)
        pltpu.async_copy(lin_hbm.at[idxf_v.at[f]], lin_v.at[f], sem_l)
        return carry

    lax.fori_loop(0, FH, fire, 0)
    pltpu.make_async_copy(int_hbm.at[:, :, pl.ds(0, BPW)], cols_v,
                          sem_c).wait()

    def drain_lin(f, carry):
        pltpu.make_async_copy(lin_hbm.at[pl.ds(0, BPW)], lin_v.at[f],
                              sem_l).wait()
        return carry

    lax.fori_loop(0, FH, drain_lin, 0)

    for c in range(NCH):
        acc = jnp.zeros((L,), jnp.float32)
        for f in range(FH):
            acc = acc + lin_v[f, pl.ds(c * L, L)]
        l_v[pl.ds(c * L, L)] = acc

    zero = jnp.zeros((L,), jnp.float32)

    def chunk_body(c, carry):
        def f_body(f, acc):
            q = acc[0]
            out = [None] * (D + 1)
            for d in range(D):
                v = cols_v[f, d, pl.ds(c * L, L)]
                out[d + 1] = acc[d + 1] + v
                q = q + v * v
            out[0] = q
            return tuple(out)

        acc = lax.fori_loop(0, FH, f_body, (zero,) * (D + 1))
        q_v[pl.ds(c * L, L)] = acc[0]
        for d in range(D):
            s_v[d, pl.ds(c * L, L)] = acc[d + 1]
        return carry

    lax.fori_loop(0, NCH, chunk_body, 0)

    pltpu.sync_copy(s_v, s_out.at[:, pl.ds(base, BPW)])
    pltpu.sync_copy(q_v, q_out.at[pl.ds(base, BPW)])
    pltpu.sync_copy(l_v, l_out.at[pl.ds(base, BPW)])


_sc_gather = functools.partial(
    pl.kernel,
    out_type=[
        jax.ShapeDtypeStruct((D, B), jnp.float32),
        jax.ShapeDtypeStruct((B,), jnp.float32),
        jax.ShapeDtypeStruct((B,), jnp.float32),
    ],
    mesh=plsc.VectorSubcoreMesh(core_axis_name="c", subcore_axis_name="s"),
    compiler_params=pltpu.CompilerParams(use_tc_tiling_on_sc=False),
    scratch_types=[
        pltpu.VMEM((FH, BPW), jnp.int32),
        pltpu.VMEM((FH, BPW), jnp.int32),
        pltpu.VMEM((FH, D, BPW), jnp.float32),
        pltpu.VMEM((FH, BPW), jnp.float32),
        pltpu.VMEM((D, BPW), jnp.float32),
        pltpu.VMEM((BPW,), jnp.float32),
        pltpu.VMEM((BPW,), jnp.float32),
        pltpu.SemaphoreType.DMA,
        pltpu.SemaphoreType.DMA,
    ],
)(_sc_body)


def _tc_body(sa_ref, qa_ref, la_ref, sb_ref, qb_ref, lb_ref,
             xn_ref, nv_ref, b_ref, o_ref):
    xn = xn_ref[...]
    nv = nv_ref[...]
    snum = lax.dot_general(nv, xn, (((0,), (1,)), ((), ())),
                           preferred_element_type=jnp.float32,
                           precision=lax.Precision.HIGHEST)
    S = sa_ref[...] + sb_ref[...] + snum
    qnum = lax.dot_general(nv * nv, xn * xn, (((0,), (1,)), ((), ())),
                           preferred_element_type=jnp.float32,
                           precision=lax.Precision.HIGHEST)
    inter = 0.5 * (jnp.sum(S * S - qnum, axis=0, keepdims=True)
                   - qa_ref[...] - qb_ref[...])
    o_ref[...] = la_ref[...] + lb_ref[...] + b_ref[0] + inter


_tc_combine = pl.pallas_call(
    _tc_body,
    out_shape=jax.ShapeDtypeStruct((1, B), jnp.float32),
    in_specs=[pl.BlockSpec(memory_space=pltpu.VMEM)] * 8
    + [pl.BlockSpec(memory_space=pltpu.SMEM)],
)


def kernel(x_numeric, x_categorical, lin_tables, int_tables, num_vecs, bias):
    int_t = jnp.transpose(int_tables, (0, 2, 1))
    lin_flat = lin_tables.reshape(F * V)
    offs = (jnp.arange(F, dtype=jnp.int32) * V)[None, :]
    flat_idx = x_categorical + offs

    def half(lo):
        idx3 = (x_categorical[:, lo:lo + FH]
                .reshape(NW, BPW, FH).transpose(0, 2, 1))
        idx3f = (flat_idx[:, lo:lo + FH]
                 .reshape(NW, BPW, FH).transpose(0, 2, 1))
        return _sc_gather(idx3, idx3f, int_t[lo:lo + FH], lin_flat)

    s_a, q_a, l_a = half(0)
    s_b, q_b, l_b = half(FH)
    out = _tc_combine(s_a, q_a[None, :], l_a[None, :],
                      s_b, q_b[None, :], l_b[None, :],
                      x_numeric, num_vecs, bias)
    return out[0]


# speedup vs baseline: 1.2103x; 1.2103x over previous
import functools

import jax
import jax.numpy as jnp
from jax import lax
from jax.experimental import pallas as pl
from jax.experimental.pallas import tpu as pltpu
from jax.experimental.pallas import tpu_sc as plsc

B = 4096
F = 26
V = 100000
D = 32
NN = 13

NC = 2
NS = 16
NW = NC * NS
BPW = B // NW
L = 16
NCH = BPW // L


def _sc_body(idx_hbm, idxf_hbm, int_hbm, lin_hbm, s_out, q_out, l_out,
             idx_v, idxf_v, cols_v, lin_v, s_v, q_v, l_v, sem_c, sem_l):
    wid = lax.axis_index("s") * NC + lax.axis_index("c")
    base = wid * BPW
    pltpu.sync_copy(idx_hbm.at[wid], idx_v)
    pltpu.sync_copy(idxf_hbm.at[wid], idxf_v)

    def fire(f, carry):
        for d in range(D):
            pltpu.async_copy(int_hbm.at[f, d].at[idx_v.at[f]],
                             cols_v.at[f, d], sem_c)
        pltpu.async_copy(lin_hbm.at[idxf_v.at[f]], lin_v.at[f], sem_l)
        return carry

    lax.fori_loop(0, F, fire, 0)
    pltpu.make_async_copy(int_hbm.at[:, :, pl.ds(0, BPW)], cols_v,
                          sem_c).wait()

    def drain_lin(f, carry):
        pltpu.make_async_copy(lin_hbm.at[pl.ds(0, BPW)], lin_v.at[f],
                              sem_l).wait()
        return carry

    lax.fori_loop(0, F, drain_lin, 0)

    for c in range(NCH):
        acc = jnp.zeros((L,), jnp.float32)
        for f in range(F):
            acc = acc + lin_v[f, pl.ds(c * L, L)]
        l_v[pl.ds(c * L, L)] = acc

    zero = jnp.zeros((L,), jnp.float32)

    def chunk_body(c, carry):
        def f_body(f, acc):
            q = acc[0]
            out = [None] * (D + 1)
            for d in range(D):
                v = cols_v[f, d, pl.ds(c * L, L)]
                out[d + 1] = acc[d + 1] + v
                q = q + v * v
            out[0] = q
            return tuple(out)

        acc = lax.fori_loop(0, F, f_body, (zero,) * (D + 1))
        q_v[pl.ds(c * L, L)] = acc[0]
        for d in range(D):
            s_v[d, pl.ds(c * L, L)] = acc[d + 1]
        return carry

    lax.fori_loop(0, NCH, chunk_body, 0)

    pltpu.sync_copy(s_v, s_out.at[:, pl.ds(base, BPW)])
    pltpu.sync_copy(q_v, q_out.at[pl.ds(base, BPW)])
    pltpu.sync_copy(l_v, l_out.at[pl.ds(base, BPW)])


_sc_gather = functools.partial(
    pl.kernel,
    out_type=[
        jax.ShapeDtypeStruct((D, B), jnp.float32),
        jax.ShapeDtypeStruct((B,), jnp.float32),
        jax.ShapeDtypeStruct((B,), jnp.float32),
    ],
    mesh=plsc.VectorSubcoreMesh(core_axis_name="c", subcore_axis_name="s"),
    compiler_params=pltpu.CompilerParams(use_tc_tiling_on_sc=False),
    scratch_types=[
        pltpu.VMEM((F, BPW), jnp.int32),
        pltpu.VMEM((F, BPW), jnp.int32),
        pltpu.VMEM((F, D, BPW), jnp.float32),
        pltpu.VMEM((F, BPW), jnp.float32),
        pltpu.VMEM((D, BPW), jnp.float32),
        pltpu.VMEM((BPW,), jnp.float32),
        pltpu.VMEM((BPW,), jnp.float32),
        pltpu.SemaphoreType.DMA,
        pltpu.SemaphoreType.DMA,
    ],
)(_sc_body)


def _tc_body(s_ref, q_ref, l_ref, xn_ref, nv_ref, b_ref, o_ref):
    xn = xn_ref[...]
    nv = nv_ref[...]
    snum = lax.dot_general(nv, xn, (((0,), (1,)), ((), ())),
                           preferred_element_type=jnp.float32,
                           precision=lax.Precision.HIGHEST)
    S = s_ref[...] + snum
    qnum = lax.dot_general(nv * nv, xn * xn, (((0,), (1,)), ((), ())),
                           preferred_element_type=jnp.float32,
                           precision=lax.Precision.HIGHEST)
    inter = 0.5 * (jnp.sum(S * S - qnum, axis=0, keepdims=True)
                   - q_ref[...])
    o_ref[...] = l_ref[...] + b_ref[0] + inter


_tc_combine = pl.pallas_call(
    _tc_body,
    out_shape=jax.ShapeDtypeStruct((1, B), jnp.float32),
    in_specs=[
        pl.BlockSpec(memory_space=pltpu.VMEM),
        pl.BlockSpec(memory_space=pltpu.VMEM),
        pl.BlockSpec(memory_space=pltpu.VMEM),
        pl.BlockSpec(memory_space=pltpu.VMEM),
        pl.BlockSpec(memory_space=pltpu.VMEM),
        pl.BlockSpec(memory_space=pltpu.SMEM),
    ],
)


def kernel(x_numeric, x_categorical, lin_tables, int_tables, num_vecs, bias):
    int_t = jnp.transpose(int_tables, (0, 2, 1))
    lin_flat = lin_tables.reshape(F * V)
    offs = (jnp.arange(F, dtype=jnp.int32) * V)[None, :]
    flat_idx = x_categorical + offs
    idx3 = x_categorical.reshape(NW, BPW, F).transpose(0, 2, 1)
    idx3f = flat_idx.reshape(NW, BPW, F).transpose(0, 2, 1)
    s_t, q_cat, lin_sum = _sc_gather(idx3, idx3f, int_t, lin_flat)
    out = _tc_combine(s_t, q_cat[None, :], lin_sum[None, :], x_numeric,
                      num_vecs, bias)
    return out[0]
